# baseline (speedup 1.0000x reference)
import functools

import jax
import jax.numpy as jnp
from jax import lax
from jax.experimental import pallas as pl
from jax.experimental.pallas import tpu as pltpu
from jax.experimental.pallas import tpu_sc as plsc

C = 256
LANES = 16
OH = OW = 7
BINS = OH * OW
ROWS_PER_BIN = 16
ROWS_PER_BOX = BINS * ROWS_PER_BIN
CHUNK_BINS = 7
CHUNK_ROWS = CHUNK_BINS * ROWS_PER_BIN
NCHUNKS = BINS // CHUNK_BINS
NCORES = 2
NSUBCORES = 16
NTILES = NCORES * NSUBCORES
BOX_PER_TILE_DEFAULT = 32

SIZES = (256, 128, 64, 32)
SCALES = (0.25, 0.125, 0.0625, 0.03125)
OFFSETS = (0, 65536, 81920, 86016)
LEVEL_THRESH = (112.0 * 112.0, 224.0 * 224.0, 448.0 * 448.0)


def _sc_roialign(table, boxes_pad, box_per_tile):
    nbox_pad = boxes_pad.shape[0] // 4
    mesh = plsc.VectorSubcoreMesh(core_axis_name="c", subcore_axis_name="s")

    @functools.partial(
        pl.kernel,
        mesh=mesh,
        compiler_params=pltpu.CompilerParams(needs_layout_passes=False),
        out_type=jax.ShapeDtypeStruct((nbox_pad, C * BINS), jnp.float32),
        scratch_types=[
            pltpu.VMEM((box_per_tile * 4,), jnp.float32),
            pltpu.VMEM((ROWS_PER_BOX,), jnp.int32),
            pltpu.VMEM((ROWS_PER_BOX,), jnp.float32),
            pltpu.VMEM((CHUNK_ROWS, C // 2), jnp.int32),
            pltpu.VMEM((CHUNK_ROWS, C // 2), jnp.int32),
            pltpu.VMEM((2 * C * BINS,), jnp.float32),
            pltpu.SemaphoreType.DMA,
            pltpu.SemaphoreType.DMA,
            pltpu.SemaphoreType.DMA,
            pltpu.SemaphoreType.DMA,
        ],
    )
    def k(table_hbm, boxes_hbm, out_hbm, box_v, idx_v, w_v, rows_a, rows_b,
          acc_t, sem_a, sem_b, sem_oa, sem_ob):
        wid = lax.axis_index("s") * NCORES + lax.axis_index("c")
        box0 = wid * box_per_tile
        pltpu.sync_copy(boxes_hbm.at[pl.ds(box0 * 4, box_per_tile * 4)], box_v)

        lane = lax.iota(jnp.int32, LANES)
        dyf = ((lane >> 3) & 1).astype(jnp.float32)
        dxf = ((lane >> 2) & 1).astype(jnp.float32)
        cyb = (lane >> 1) & 1
        cxb = lane & 1
        cy_hi = cyb == 1
        cx_hi = cxb == 1

        iota98 = lane * (2 * BINS)
        cb = C * BINS

        def full_i(x):
            return jnp.full((LANES,), x, jnp.int32)

        def drain_out(half, sem):
            pltpu.make_async_copy(
                out_hbm.at[0], acc_t.at[pl.ds(half * cb, cb)], sem).wait()

        def do_box(b, carry):
            b4 = b * 4
            x1 = plsc.load_gather(box_v, [full_i(b4)])
            y1 = plsc.load_gather(box_v, [full_i(b4 + 1)])
            x2 = plsc.load_gather(box_v, [full_i(b4 + 2)])
            y2 = plsc.load_gather(box_v, [full_i(b4 + 3)])

            wh = (x2 - x1) * (y2 - y1)
            lvl = (
                (wh >= LEVEL_THRESH[0]).astype(jnp.int32)
                + (wh >= LEVEL_THRESH[1]).astype(jnp.int32)
                + (wh >= LEVEL_THRESH[2]).astype(jnp.int32)
            )
            is1 = lvl == 1
            is2 = lvl == 2
            is3 = lvl == 3
            scale = jnp.where(
                is3, SCALES[3],
                jnp.where(is2, SCALES[2], jnp.where(is1, SCALES[1], SCALES[0])))
            wi = jnp.where(
                is3, SIZES[3],
                jnp.where(is2, SIZES[2],
                          jnp.where(is1, SIZES[1], SIZES[0]))).astype(jnp.int32)
            offi = jnp.where(
                is3, OFFSETS[3],
                jnp.where(is2, OFFSETS[2],
                          jnp.where(is1, OFFSETS[1],
                                    OFFSETS[0]))).astype(jnp.int32)
            wf = wi.astype(jnp.float32)
            wm1f = wf - 1.0
            wm1i = wi - 1

            x1s = x1 * scale
            y1s = y1 * scale
            roi_w = jnp.maximum(x2 * scale - x1s, 1.0)
            roi_h = jnp.maximum(y2 * scale - y1s, 1.0)
            bin_w = roi_w / 7.0
            bin_h = roi_h / 7.0

            def do_bin(bn, carry2):
                by = bn // OW
                bx = bn - by * OW
                gy = (2.0 * by.astype(jnp.float32) + dyf + 0.5) * 0.5
                gx = (2.0 * bx.astype(jnp.float32) + dxf + 0.5) * 0.5
                yy = y1s + gy * bin_h
                xx = x1s + gx * bin_w
                valid = (yy > -1.0) & (yy < wf) & (xx > -1.0) & (xx < wf)
                yc = jnp.minimum(jnp.maximum(yy, 0.0), wm1f)
                xc = jnp.minimum(jnp.maximum(xx, 0.0), wm1f)
                y0i = yc.astype(jnp.int32)
                x0i = xc.astype(jnp.int32)
                ly = yc - y0i.astype(jnp.float32)
                lx = xc - x0i.astype(jnp.float32)
                yci = jnp.minimum(y0i + cyb, wm1i)
                xci = jnp.minimum(x0i + cxb, wm1i)
                wy = jnp.where(cy_hi, ly, 1.0 - ly)
                wx = jnp.where(cx_hi, lx, 1.0 - lx)
                wgt = jnp.where(valid, wy * wx * 0.25, 0.0)
                idx = offi + yci * wi + xci
                idx_v[pl.ds(bn * ROWS_PER_BIN, LANES)] = idx
                w_v[pl.ds(bn * ROWS_PER_BIN, LANES)] = wgt
                return carry2

            sel_off = (b % 2) * cb

            @pl.when(b >= 2)
            def _():
                lax.cond(b % 2 == 0,
                         lambda: drain_out(0, sem_oa),
                         lambda: drain_out(1, sem_ob))

            bufs = (rows_a, rows_b)
            sems = (sem_a, sem_b)

            def fire(c):
                return pltpu.async_copy(
                    table_hbm.at[idx_v.at[pl.ds(c * CHUNK_ROWS, CHUNK_ROWS)]],
                    bufs[c % 2], sems[c % 2])

            fired = []
            for c in range(NCHUNKS):
                lax.fori_loop(c * CHUNK_BINS, (c + 1) * CHUNK_BINS, do_bin, 0)
                if c < 2:
                    fired.append(fire(c))

            descs = fired + [None] * (NCHUNKS - 2)
            for c in range(NCHUNKS):
                descs[c].wait()
                rows = bufs[c % 2]

                def do_bin_acc(r, carry2, _c=c, _rows=rows):
                    bn = _c * CHUNK_BINS + r
                    rbase = r * ROWS_PER_BIN
                    wbase = bn * ROWS_PER_BIN
                    acc = [jnp.zeros((LANES,), jnp.float32)
                           for _ in range(C // LANES)]
                    for j2 in range(ROWS_PER_BIN // 2):
                        wb0 = plsc.load_gather(w_v, [full_i(wbase + 2 * j2)])
                        wb1 = plsc.load_gather(
                            w_v, [full_i(wbase + 2 * j2 + 1)])
                        wp0 = plsc.pack(wb0, wb0,
                                        format=plsc.PackFormat.INTERLEAVED)
                        wp1 = plsc.pack(wb1, wb1,
                                        format=plsc.PackFormat.INTERLEAVED)
                        r0 = rbase + 2 * j2
                        for t in range(C // 32):
                            v0 = plsc.bitcast(
                                _rows[r0, pl.ds(t * LANES, LANES)],
                                jnp.bfloat16)
                            v1 = plsc.bitcast(
                                _rows[r0 + 1, pl.ds(t * LANES, LANES)],
                                jnp.bfloat16)
                            prod = wp0 * v0 + wp1 * v1
                            ea, eb = plsc.unpack(
                                prod, format=plsc.PackFormat.INTERLEAVED)
                            acc[t * 2] = acc[t * 2] + ea
                            acc[t * 2 + 1] = acc[t * 2 + 1] + eb
                    base_s = sel_off + bn
                    for t in range(C // 32):
                        plsc.store_scatter(
                            acc_t, [iota98 + (base_s + t * 32 * BINS)],
                            acc[2 * t])
                        plsc.store_scatter(
                            acc_t,
                            [iota98 + (base_s + (t * 32 + 1) * BINS)],
                            acc[2 * t + 1])
                    return carry2

                lax.fori_loop(0, CHUNK_BINS, do_bin_acc, 0)
                if c + 2 < NCHUNKS:
                    descs[c + 2] = fire(c + 2)

            lax.cond(
                b % 2 == 0,
                lambda: pltpu.async_copy(
                    acc_t.at[pl.ds(0, cb)], out_hbm.at[box0 + b],
                    sem_oa) and None,
                lambda: pltpu.async_copy(
                    acc_t.at[pl.ds(cb, cb)], out_hbm.at[box0 + b],
                    sem_ob) and None)
            return carry

        lax.fori_loop(0, box_per_tile, do_box, 0)
        drain_out(0, sem_oa)
        if box_per_tile >= 2:
            drain_out(1, sem_ob)

    return k(table, boxes_pad)


def kernel(feat0, feat1, feat2, feat3, boxes, image_shapes):
    del image_shapes

    def rows(f):
        _, c, h, w = f.shape
        r = jnp.transpose(f.reshape(c, h * w), (1, 0)).astype(jnp.bfloat16)
        return jax.lax.bitcast_convert_type(
            r.reshape(h * w, C // 2, 2), jnp.int32)

    table = jnp.concatenate([rows(feat0), rows(feat1), rows(feat2),
                             rows(feat3)], axis=0)
    n = boxes.shape[0]
    pad = (-n) % NTILES
    boxes_pad = jnp.pad(boxes, ((0, pad), (0, 0))).reshape(-1)
    box_per_tile = (n + pad) // NTILES
    out = _sc_roialign(table, boxes_pad, box_per_tile)
    return out[:n].reshape(n, C, OH, OW)


# speedup vs baseline: 1.2834x; 1.2834x over previous
import functools

import jax
import jax.numpy as jnp
from jax import lax
from jax.experimental import pallas as pl
from jax.experimental.pallas import tpu as pltpu
from jax.experimental.pallas import tpu_sc as plsc

C = 256
LANES = 16
OH = OW = 7
BINS = OH * OW
ROWS_PER_BIN = 16
ROWS_PER_BOX = BINS * ROWS_PER_BIN
CHUNK_BINS = 7
CHUNK_ROWS = CHUNK_BINS * ROWS_PER_BIN
NCHUNKS = BINS // CHUNK_BINS
NCORES = 2
NSUBCORES = 16
NTILES = NCORES * NSUBCORES
BOX_PER_TILE_DEFAULT = 32

SIZES = (256, 128, 64, 32)
SCALES = (0.25, 0.125, 0.0625, 0.03125)
OFFSETS = (0, 65536, 81920, 86016)
LEVEL_THRESH = (112.0 * 112.0, 224.0 * 224.0, 448.0 * 448.0)


def _sc_roialign(table, boxes_pad, nbox_pad, bpt0, bpt1):
    bmax = max(bpt0, bpt1)
    mesh = plsc.VectorSubcoreMesh(core_axis_name="c", subcore_axis_name="s")

    @functools.partial(
        pl.kernel,
        mesh=mesh,
        compiler_params=pltpu.CompilerParams(needs_layout_passes=False),
        out_type=jax.ShapeDtypeStruct((nbox_pad, C * BINS), jnp.float32),
        scratch_types=[
            pltpu.VMEM((bmax * 4,), jnp.float32),
            pltpu.VMEM((ROWS_PER_BOX,), jnp.int32),
            pltpu.VMEM((ROWS_PER_BOX,), jnp.float32),
            pltpu.VMEM((CHUNK_ROWS, C // 2), jnp.int32),
            pltpu.VMEM((CHUNK_ROWS, C // 2), jnp.int32),
            pltpu.VMEM((2 * C * BINS,), jnp.float32),
            pltpu.SemaphoreType.DMA,
            pltpu.SemaphoreType.DMA,
            pltpu.SemaphoreType.DMA,
            pltpu.SemaphoreType.DMA,
        ],
    )
    def k(table_hbm, boxes_hbm, out_hbm, box_v, idx_v, w_v, rows_a, rows_b,
          acc_t, sem_a, sem_b, sem_oa, sem_ob):
        c_ax = lax.axis_index("c")
        s_ax = lax.axis_index("s")
        is_c1 = (c_ax == 1).astype(jnp.int32)
        bpt = jnp.where(is_c1 == 1, bpt1, bpt0)
        box0 = s_ax * bpt + is_c1 * (NSUBCORES * bpt0)
        pltpu.sync_copy(boxes_hbm.at[pl.ds(box0 * 4, bmax * 4)], box_v)

        lane = lax.iota(jnp.int32, LANES)
        dyf = ((lane >> 3) & 1).astype(jnp.float32)
        dxf = ((lane >> 2) & 1).astype(jnp.float32)
        cyb = (lane >> 1) & 1
        cxb = lane & 1
        cy_hi = cyb == 1
        cx_hi = cxb == 1

        iota49 = lane * BINS
        cb = C * BINS

        def full_i(x):
            return jnp.full((LANES,), x, jnp.int32)

        def drain_out(half, sem):
            pltpu.make_async_copy(
                out_hbm.at[0], acc_t.at[pl.ds(half * cb, cb)], sem).wait()

        def do_box(b, carry):
            b4 = b * 4
            x1 = plsc.load_gather(box_v, [full_i(b4)])
            y1 = plsc.load_gather(box_v, [full_i(b4 + 1)])
            x2 = plsc.load_gather(box_v, [full_i(b4 + 2)])
            y2 = plsc.load_gather(box_v, [full_i(b4 + 3)])

            wh = (x2 - x1) * (y2 - y1)
            lvl = (
                (wh >= LEVEL_THRESH[0]).astype(jnp.int32)
                + (wh >= LEVEL_THRESH[1]).astype(jnp.int32)
                + (wh >= LEVEL_THRESH[2]).astype(jnp.int32)
            )
            is1 = lvl == 1
            is2 = lvl == 2
            is3 = lvl == 3
            scale = jnp.where(
                is3, SCALES[3],
                jnp.where(is2, SCALES[2], jnp.where(is1, SCALES[1], SCALES[0])))
            wi = jnp.where(
                is3, SIZES[3],
                jnp.where(is2, SIZES[2],
                          jnp.where(is1, SIZES[1], SIZES[0]))).astype(jnp.int32)
            offi = jnp.where(
                is3, OFFSETS[3],
                jnp.where(is2, OFFSETS[2],
                          jnp.where(is1, OFFSETS[1],
                                    OFFSETS[0]))).astype(jnp.int32)
            wf = wi.astype(jnp.float32)
            wm1f = wf - 1.0
            wm1i = wi - 1

            x1s = x1 * scale
            y1s = y1 * scale
            roi_w = jnp.maximum(x2 * scale - x1s, 1.0)
            roi_h = jnp.maximum(y2 * scale - y1s, 1.0)
            bin_w = roi_w / 7.0
            bin_h = roi_h / 7.0

            def do_bin(bn, carry2):
                by = bn // OW
                bx = bn - by * OW
                gy = (2.0 * by.astype(jnp.float32) + dyf + 0.5) * 0.5
                gx = (2.0 * bx.astype(jnp.float32) + dxf + 0.5) * 0.5
                yy = y1s + gy * bin_h
                xx = x1s + gx * bin_w
                valid = (yy > -1.0) & (yy < wf) & (xx > -1.0) & (xx < wf)
                yc = jnp.minimum(jnp.maximum(yy, 0.0), wm1f)
                xc = jnp.minimum(jnp.maximum(xx, 0.0), wm1f)
                y0i = yc.astype(jnp.int32)
                x0i = xc.astype(jnp.int32)
                ly = yc - y0i.astype(jnp.float32)
                lx = xc - x0i.astype(jnp.float32)
                yci = jnp.minimum(y0i + cyb, wm1i)
                xci = jnp.minimum(x0i + cxb, wm1i)
                wy = jnp.where(cy_hi, ly, 1.0 - ly)
                wx = jnp.where(cx_hi, lx, 1.0 - lx)
                wgt = jnp.where(valid, wy * wx * 0.25, 0.0)
                idx = offi + yci * wi + xci
                idx_v[pl.ds(bn * ROWS_PER_BIN, LANES)] = idx
                w_v[pl.ds(bn * ROWS_PER_BIN, LANES)] = wgt
                return carry2

            sel_off = (b % 2) * cb

            @pl.when(b >= 2)
            def _():
                lax.cond(b % 2 == 0,
                         lambda: drain_out(0, sem_oa),
                         lambda: drain_out(1, sem_ob))

            bufs = (rows_a, rows_b)
            sems = (sem_a, sem_b)

            def fire(c):
                return pltpu.async_copy(
                    table_hbm.at[idx_v.at[pl.ds(c * CHUNK_ROWS, CHUNK_ROWS)]],
                    bufs[c % 2], sems[c % 2])

            fired = []
            for c in range(NCHUNKS):
                lax.fori_loop(c * CHUNK_BINS, (c + 1) * CHUNK_BINS, do_bin, 0)
                if c < 2:
                    fired.append(fire(c))

            descs = fired + [None] * (NCHUNKS - 2)
            for c in range(NCHUNKS):
                descs[c].wait()
                rows = bufs[c % 2]

                def do_bin_acc(r, carry2, _c=c, _rows=rows):
                    bn = _c * CHUNK_BINS + r
                    rbase = r * ROWS_PER_BIN
                    wbase = bn * ROWS_PER_BIN
                    acc = [jnp.zeros((LANES,), jnp.float32)
                           for _ in range(C // LANES)]
                    for j2 in range(ROWS_PER_BIN // 2):
                        wb0 = plsc.load_gather(w_v, [full_i(wbase + 2 * j2)])
                        wb1 = plsc.load_gather(
                            w_v, [full_i(wbase + 2 * j2 + 1)])
                        wp0 = plsc.pack(wb0, wb0,
                                        format=plsc.PackFormat.INTERLEAVED)
                        wp1 = plsc.pack(wb1, wb1,
                                        format=plsc.PackFormat.INTERLEAVED)
                        r0 = rbase + 2 * j2
                        for t in range(C // 32):
                            v0 = plsc.bitcast(
                                _rows[r0, pl.ds(t * LANES, LANES)],
                                jnp.bfloat16)
                            v1 = plsc.bitcast(
                                _rows[r0 + 1, pl.ds(t * LANES, LANES)],
                                jnp.bfloat16)
                            prod = wp0 * v0 + wp1 * v1
                            ea, eb = plsc.unpack(
                                prod, format=plsc.PackFormat.INTERLEAVED)
                            acc[t * 2] = acc[t * 2] + ea
                            acc[t * 2 + 1] = acc[t * 2 + 1] + eb
                    base_s = sel_off + bn
                    for ch in range(C // LANES):
                        plsc.store_scatter(
                            acc_t, [iota49 + (base_s + ch * LANES * BINS)],
                            acc[ch])
                    return carry2

                lax.fori_loop(0, CHUNK_BINS, do_bin_acc, 0)
                if c + 2 < NCHUNKS:
                    descs[c + 2] = fire(c + 2)

            lax.cond(
                b % 2 == 0,
                lambda: pltpu.async_copy(
                    acc_t.at[pl.ds(0, cb)], out_hbm.at[box0 + b],
                    sem_oa) and None,
                lambda: pltpu.async_copy(
                    acc_t.at[pl.ds(cb, cb)], out_hbm.at[box0 + b],
                    sem_ob) and None)
            return carry

        lax.fori_loop(0, bpt, do_box, 0)
        drain_out(0, sem_oa)
        drain_out(1, sem_ob)

    return k(table, boxes_pad)


def kernel(feat0, feat1, feat2, feat3, boxes, image_shapes):
    del image_shapes

    def rows(f):
        _, c, h, w = f.shape
        r = jnp.transpose(f.reshape(c, h * w), (1, 0))
        r = r.reshape(h * w, 8, 2, 16).transpose(0, 1, 3, 2)
        r = r.astype(jnp.bfloat16).reshape(h * w, C // 2, 2)
        return jax.lax.bitcast_convert_type(r, jnp.int32)

    table = jnp.concatenate([rows(feat0), rows(feat1), rows(feat2),
                             rows(feat3)], axis=0)
    n = boxes.shape[0]
    total = ((n + NTILES - 1) // NTILES) * NTILES
    base = total // NTILES
    delta = min(6, base - 1)
    bpt0, bpt1 = base + delta, base - delta
    bmax = max(bpt0, bpt1)
    boxes_pad = jnp.pad(boxes, ((0, total + bmax - n), (0, 0))).reshape(-1)
    out = _sc_roialign(table, boxes_pad, total, bpt0, bpt1)
    return out[:n].reshape(n, C, OH, OW)


# speedup vs baseline: 1.3067x; 1.0182x over previous
import functools

import jax
import jax.numpy as jnp
from jax import lax
from jax.experimental import pallas as pl
from jax.experimental.pallas import tpu as pltpu
from jax.experimental.pallas import tpu_sc as plsc

C = 256
LANES = 16
OH = OW = 7
BINS = OH * OW
ROWS_PER_BIN = 16
ROWS_PER_BOX = BINS * ROWS_PER_BIN
CHUNK_BINS = 7
CHUNK_ROWS = CHUNK_BINS * ROWS_PER_BIN
NCHUNKS = BINS // CHUNK_BINS
NCORES = 2
NSUBCORES = 16
NTILES = NCORES * NSUBCORES
BOX_PER_TILE_DEFAULT = 32

SIZES = (256, 128, 64, 32)
SCALES = (0.25, 0.125, 0.0625, 0.03125)
OFFSETS = (0, 65536, 81920, 86016)
LEVEL_THRESH = (112.0 * 112.0, 224.0 * 224.0, 448.0 * 448.0)


def _sc_roialign(table, boxes_pad, nbox_pad, bpt0, bpt1):
    bmax = max(bpt0, bpt1)
    mesh = plsc.VectorSubcoreMesh(core_axis_name="c", subcore_axis_name="s")

    @functools.partial(
        pl.kernel,
        mesh=mesh,
        compiler_params=pltpu.CompilerParams(needs_layout_passes=False),
        out_type=jax.ShapeDtypeStruct((nbox_pad, C * BINS), jnp.float32),
        scratch_types=[
            pltpu.VMEM((bmax * 4,), jnp.float32),
            pltpu.VMEM((ROWS_PER_BOX,), jnp.int32),
            pltpu.VMEM((ROWS_PER_BOX,), jnp.float32),
            pltpu.VMEM((CHUNK_ROWS, C // 2), jnp.int32),
            pltpu.VMEM((CHUNK_ROWS, C // 2), jnp.int32),
            pltpu.VMEM((2 * C * BINS,), jnp.float32),
            pltpu.SemaphoreType.DMA,
            pltpu.SemaphoreType.DMA,
            pltpu.SemaphoreType.DMA,
            pltpu.SemaphoreType.DMA,
        ],
    )
    def k(table_hbm, boxes_hbm, out_hbm, box_v, idx_v, w_v, rows_a, rows_b,
          acc_t, sem_a, sem_b, sem_oa, sem_ob):
        c_ax = lax.axis_index("c")
        s_ax = lax.axis_index("s")
        is_c1 = (c_ax == 1).astype(jnp.int32)
        bpt = jnp.where(is_c1 == 1, bpt1, bpt0)
        box0 = s_ax * bpt + is_c1 * (NSUBCORES * bpt0)
        pltpu.sync_copy(boxes_hbm.at[pl.ds(box0 * 4, bmax * 4)], box_v)

        lane = lax.iota(jnp.int32, LANES)
        dyf = ((lane >> 3) & 1).astype(jnp.float32)
        dxf = ((lane >> 2) & 1).astype(jnp.float32)
        cyb = (lane >> 1) & 1
        cxb = lane & 1
        cy_hi = cyb == 1
        cx_hi = cxb == 1

        iota49 = lane * BINS
        cb = C * BINS

        def full_i(x):
            return jnp.full((LANES,), x, jnp.int32)

        def drain_out(half, sem):
            pltpu.make_async_copy(
                out_hbm.at[0], acc_t.at[pl.ds(half * cb, cb)], sem).wait()

        def do_box(b, carry):
            b4 = b * 4
            x1 = plsc.load_gather(box_v, [full_i(b4)])
            y1 = plsc.load_gather(box_v, [full_i(b4 + 1)])
            x2 = plsc.load_gather(box_v, [full_i(b4 + 2)])
            y2 = plsc.load_gather(box_v, [full_i(b4 + 3)])

            wh = (x2 - x1) * (y2 - y1)
            lvl = (
                (wh >= LEVEL_THRESH[0]).astype(jnp.int32)
                + (wh >= LEVEL_THRESH[1]).astype(jnp.int32)
                + (wh >= LEVEL_THRESH[2]).astype(jnp.int32)
            )
            is1 = lvl == 1
            is2 = lvl == 2
            is3 = lvl == 3
            scale = jnp.where(
                is3, SCALES[3],
                jnp.where(is2, SCALES[2], jnp.where(is1, SCALES[1], SCALES[0])))
            wi = jnp.where(
                is3, SIZES[3],
                jnp.where(is2, SIZES[2],
                          jnp.where(is1, SIZES[1], SIZES[0]))).astype(jnp.int32)
            offi = jnp.where(
                is3, OFFSETS[3],
                jnp.where(is2, OFFSETS[2],
                          jnp.where(is1, OFFSETS[1],
                                    OFFSETS[0]))).astype(jnp.int32)
            wf = wi.astype(jnp.float32)
            wm1f = wf - 1.0
            wm1i = wi - 1

            x1s = x1 * scale
            y1s = y1 * scale
            roi_w = jnp.maximum(x2 * scale - x1s, 1.0)
            roi_h = jnp.maximum(y2 * scale - y1s, 1.0)
            bin_w = roi_w / 7.0
            bin_h = roi_h / 7.0

            def do_bin(bn, carry2):
                by = bn // OW
                bx = bn - by * OW
                gy = (2.0 * by.astype(jnp.float32) + dyf + 0.5) * 0.5
                gx = (2.0 * bx.astype(jnp.float32) + dxf + 0.5) * 0.5
                yy = y1s + gy * bin_h
                xx = x1s + gx * bin_w
                valid = (yy > -1.0) & (yy < wf) & (xx > -1.0) & (xx < wf)
                yc = jnp.minimum(jnp.maximum(yy, 0.0), wm1f)
                xc = jnp.minimum(jnp.maximum(xx, 0.0), wm1f)
                y0i = yc.astype(jnp.int32)
                x0i = xc.astype(jnp.int32)
                ly = yc - y0i.astype(jnp.float32)
                lx = xc - x0i.astype(jnp.float32)
                yci = jnp.minimum(y0i + cyb, wm1i)
                xci = jnp.minimum(x0i + cxb, wm1i)
                wy = jnp.where(cy_hi, ly, 1.0 - ly)
                wx = jnp.where(cx_hi, lx, 1.0 - lx)
                wgt = jnp.where(valid, wy * wx * 0.25, 0.0)
                idx = offi + yci * wi + xci
                idx_v[pl.ds(bn * ROWS_PER_BIN, LANES)] = idx
                w_v[pl.ds(bn * ROWS_PER_BIN, LANES)] = wgt
                return carry2

            sel_off = (b % 2) * cb

            @pl.when(b >= 2)
            def _():
                lax.cond(b % 2 == 0,
                         lambda: drain_out(0, sem_oa),
                         lambda: drain_out(1, sem_ob))

            bufs = (rows_a, rows_b)
            sems = (sem_a, sem_b)

            def fire(c):
                return pltpu.async_copy(
                    table_hbm.at[idx_v.at[pl.ds(c * CHUNK_ROWS, CHUNK_ROWS)]],
                    bufs[c % 2], sems[c % 2])

            fired = []
            for c in range(NCHUNKS):
                lax.fori_loop(c * CHUNK_BINS, (c + 1) * CHUNK_BINS, do_bin, 0)
                if c < 2:
                    fired.append(fire(c))

            descs = fired + [None] * (NCHUNKS - 2)
            for c in range(NCHUNKS):
                descs[c].wait()
                rows = bufs[c % 2]

                def do_bin_acc(r, carry2, _c=c, _rows=rows):
                    bn = _c * CHUNK_BINS + r
                    rbase = r * ROWS_PER_BIN
                    wbase = bn * ROWS_PER_BIN
                    acc = [jnp.zeros((LANES,), jnp.float32)
                           for _ in range(C // LANES)]
                    for j2 in range(ROWS_PER_BIN // 2):
                        wb0 = plsc.load_gather(w_v, [full_i(wbase + 2 * j2)])
                        wb1 = plsc.load_gather(
                            w_v, [full_i(wbase + 2 * j2 + 1)])
                        wp0 = plsc.pack(wb0, wb0,
                                        format=plsc.PackFormat.INTERLEAVED)
                        wp1 = plsc.pack(wb1, wb1,
                                        format=plsc.PackFormat.INTERLEAVED)
                        r0 = rbase + 2 * j2
                        for t in range(C // 32):
                            v0 = plsc.bitcast(
                                _rows[r0, pl.ds(t * LANES, LANES)],
                                jnp.bfloat16)
                            v1 = plsc.bitcast(
                                _rows[r0 + 1, pl.ds(t * LANES, LANES)],
                                jnp.bfloat16)
                            prod = wp0 * v0 + wp1 * v1
                            ea, eb = plsc.unpack(
                                prod, format=plsc.PackFormat.INTERLEAVED)
                            acc[t * 2] = acc[t * 2] + ea
                            acc[t * 2 + 1] = acc[t * 2 + 1] + eb
                    base_s = sel_off + bn
                    for ch in range(C // LANES):
                        plsc.store_scatter(
                            acc_t, [iota49 + (base_s + ch * LANES * BINS)],
                            acc[ch])
                    return carry2

                lax.fori_loop(0, CHUNK_BINS, do_bin_acc, 0)
                if c + 2 < NCHUNKS:
                    descs[c + 2] = fire(c + 2)

            lax.cond(
                b % 2 == 0,
                lambda: pltpu.async_copy(
                    acc_t.at[pl.ds(0, cb)], out_hbm.at[box0 + b],
                    sem_oa) and None,
                lambda: pltpu.async_copy(
                    acc_t.at[pl.ds(cb, cb)], out_hbm.at[box0 + b],
                    sem_ob) and None)
            return carry

        lax.fori_loop(0, bpt, do_box, 0)
        drain_out(0, sem_oa)
        drain_out(1, sem_ob)

    return k(table, boxes_pad)


def kernel(feat0, feat1, feat2, feat3, boxes, image_shapes):
    del image_shapes

    def rows(f):
        _, c, h, w = f.shape
        r = jnp.transpose(f.reshape(c, h * w), (1, 0))
        r = r.reshape(h * w, 8, 2, 16).transpose(0, 1, 3, 2)
        r = r.astype(jnp.bfloat16).reshape(h * w, C // 2, 2)
        return jax.lax.bitcast_convert_type(r, jnp.int32)

    table = jnp.concatenate([rows(feat0), rows(feat1), rows(feat2),
                             rows(feat3)], axis=0)
    n = boxes.shape[0]
    total = ((n + NTILES - 1) // NTILES) * NTILES
    base = total // NTILES
    delta = min(8, base - 2) & ~1
    bpt0, bpt1 = base + delta, base - delta
    bmax = max(bpt0, bpt1)
    boxes_pad = jnp.pad(boxes, ((0, total + bmax - n), (0, 0))).reshape(-1)
    out = _sc_roialign(table, boxes_pad, total, bpt0, bpt1)
    return out[:n].reshape(n, C, OH, OW)
